# baseline (speedup 1.0000x reference)
import functools

import jax
import jax.numpy as jnp
from jax import lax
from jax.experimental import pallas as pl
from jax.experimental.pallas import tpu as pltpu
from jax.experimental.pallas import tpu_sc as plsc

_NC = 2
_NS = 16
_NW = _NC * _NS
_C = 128


@functools.partial(jax.jit, static_argnums=(2,))
def _gather_rows(idx, weights, nchunk):
    n = _NW * nchunk * _C
    d = weights.shape[1]
    mesh = plsc.VectorSubcoreMesh(core_axis_name="c", subcore_axis_name="s")

    @functools.partial(
        pl.kernel,
        out_type=jax.ShapeDtypeStruct((n, d), jnp.float32),
        mesh=mesh,
        scratch_types=[
            pltpu.VMEM((nchunk, _C), jnp.int32),
            pltpu.VMEM((2, _C, d), jnp.float32),
            pltpu.SemaphoreType.DMA,
            pltpu.SemaphoreType.DMA,
        ],
        compiler_params=pltpu.CompilerParams(use_tc_tiling_on_sc=False),
    )
    def k(idx_hbm, table_hbm, out_hbm, idx_v, rows_v, sem0, sem1):
        wid = lax.axis_index("s") * _NC + lax.axis_index("c")
        base = wid * (nchunk * _C)
        pltpu.sync_copy(idx_hbm.at[wid], idx_v)
        sems = (sem0, sem1)
        pltpu.async_copy(table_hbm.at[idx_v.at[0]], rows_v.at[0], sems[0])
        pltpu.async_copy(table_hbm.at[idx_v.at[1]], rows_v.at[1], sems[1])

        @pl.loop(0, nchunk - 2, step=2)
        def _(jj):
            for b in range(2):
                c = jj + b
                pltpu.make_async_copy(
                    table_hbm.at[idx_v.at[c]], rows_v.at[b], sems[b]
                ).wait()
                pltpu.sync_copy(
                    rows_v.at[b], out_hbm.at[pl.ds(base + c * _C, _C)]
                )
                pltpu.async_copy(
                    table_hbm.at[idx_v.at[c + 2]], rows_v.at[b], sems[b]
                )

        for b in range(2):
            c = nchunk - 2 + b
            pltpu.make_async_copy(
                table_hbm.at[idx_v.at[c]], rows_v.at[b], sems[b]
            ).wait()
            pltpu.sync_copy(rows_v.at[b], out_hbm.at[pl.ds(base + c * _C, _C)])

    return k(idx, weights)


def kernel(captions, weights):
    bsz, seq = captions.shape
    d = weights.shape[1]
    n = bsz * seq
    flat = captions.reshape(n).astype(jnp.int32)
    grain = _NW * _C
    n_pad = ((n + grain - 1) // grain) * grain
    if n_pad != n:
        flat = jnp.pad(flat, (0, n_pad - n))
    nchunk = n_pad // grain
    idx = flat.reshape(_NW, nchunk, _C)
    rows = _gather_rows(idx, weights, nchunk)
    return rows[:n].reshape(bsz, seq, d)


# speedup vs baseline: 1.3433x; 1.3433x over previous
import functools

import jax
import jax.numpy as jnp
from jax import lax
from jax.experimental import pallas as pl
from jax.experimental.pallas import tpu as pltpu
from jax.experimental.pallas import tpu_sc as plsc

_NC = 2
_NS = 16
_NW = _NC * _NS
_C = 128


@functools.partial(jax.jit, static_argnums=(2,))
def _gather_rows(idx, weights, nchunk):
    n = _NW * nchunk * _C
    d = weights.shape[1]
    mesh = plsc.VectorSubcoreMesh(core_axis_name="c", subcore_axis_name="s")

    @functools.partial(
        pl.kernel,
        out_type=jax.ShapeDtypeStruct((n, d), jnp.float32),
        mesh=mesh,
        scratch_types=[
            pltpu.VMEM((nchunk, _C), jnp.int32),
            pltpu.SMEM((_C,), jnp.int32),
            pltpu.VMEM((2, _C, d), jnp.float32),
            pltpu.SemaphoreType.DMA,
            pltpu.SemaphoreType.DMA,
        ],
    )
    def k(idx_hbm, table_hbm, out_hbm, idx_v, idx_s, rows_v, sem0, sem1):
        wid = lax.axis_index("s") * _NC + lax.axis_index("c")
        base = wid * (nchunk * _C)
        sems = (sem0, sem1)
        pltpu.sync_copy(idx_hbm.at[wid], idx_v)

        def fire(c, b):
            @pl.loop(0, _C, step=16)
            def _(i0):
                vec = idx_v[c, pl.ds(i0, 16)]
                for i in range(16):
                    pltpu.async_copy(
                        table_hbm.at[vec[i]],
                        rows_v.at[b, i0 + i],
                        sems[b],
                    )

        def drain_and_store(c, b):
            pltpu.make_async_copy(
                table_hbm.at[pl.ds(0, _C)], rows_v.at[b], sems[b]
            ).wait()
            pltpu.sync_copy(rows_v.at[b], out_hbm.at[pl.ds(base + c * _C, _C)])

        fire(0, 0)
        fire(1, 1)

        @pl.loop(0, nchunk - 2, step=2)
        def _(jj):
            for b in range(2):
                drain_and_store(jj + b, b)
                fire(jj + b + 2, b)

        for b in range(2):
            drain_and_store(nchunk - 2 + b, b)

    return k(idx, weights)


def kernel(captions, weights):
    bsz, seq = captions.shape
    d = weights.shape[1]
    n = bsz * seq
    flat = captions.reshape(n).astype(jnp.int32)
    grain = _NW * _C
    n_pad = ((n + grain - 1) // grain) * grain
    if n_pad != n:
        flat = jnp.pad(flat, (0, n_pad - n))
    nchunk = n_pad // grain
    idx = flat.reshape(_NW, nchunk, _C)
    rows = _gather_rows(idx, weights, nchunk)
    return rows[:n].reshape(bsz, seq, d)
